# baseline (speedup 1.0000x reference)
import jax
import jax.numpy as jnp
from jax import lax
from jax.experimental import pallas as pl
from jax.experimental.pallas import tpu as pltpu
from jax.experimental.pallas import tpu_sc as plsc

ROWS = 128
COLS = 32768
K = COLS // 4
R_POS = COLS - K + 1
R_NEG = K
L = 16
NVEC = COLS // L
NBKT = 256
NGRP = NBKT // L
NC, NS = 2, 16
NW = NC * NS
ROWS_PER_W = ROWS // NW
UNROLL = 8

INT_MIN = jnp.int32(-2147483648)
MASK31 = jnp.int32(0x7FFFFFFF)
FF = jnp.int32(0xFF)
FF0 = jnp.int32(0xFF0)


def _ukey(zv):
    bits = lax.bitcast_convert_type(zv, jnp.int32)
    s = lax.shift_right_arithmetic(bits, 31)
    skey = lax.bitwise_xor(bits, lax.bitwise_and(s, MASK31))
    return lax.bitwise_xor(skey, INT_MIN)


def _srl(x, n):
    return lax.shift_right_logical(x, jnp.int32(n))


def _shl(x, n):
    return lax.shift_left(x, jnp.int32(n))


def _clear(hist, nvals):
    zeros = jnp.zeros((L,), jnp.int32)

    @plsc.parallel_loop(0, nvals // L, unroll=UNROLL)
    def body(b):
        hist[pl.ds(b * L, L)] = zeros


def _vsearch(hist, lane16, targets, g0):
    lane = lax.iota(jnp.int32, L)
    z = jnp.int32(0)
    init = (z,) + sum(((z, z) for _ in targets), ())

    def gbody(g, carry):
        cum = carry[0]
        flat0 = g * (L * L)
        acc = jnp.zeros((L,), jnp.int32)
        for j in range(L):
            acc = acc + plsc.load_gather(hist, [lane16 + (flat0 + j)])
        cs = plsc.cumsum(acc)
        tot = jnp.max(cs)
        cum2 = cum + tot
        excl = cs - acc
        out = [cum2]
        for t, r in enumerate(targets):
            bb, base = carry[1 + 2 * t], carry[2 + 2 * t]
            hit_vec = (cum + cs) >= r
            has = jnp.logical_and(cum < r, cum2 >= r)
            idx_splat = plsc.all_reduce_ffs(hit_vec)
            prev = jnp.max(jnp.where(lane == idx_splat, excl, z))
            bb = jnp.where(has, _shl(g - g0, 4) + jnp.max(idx_splat), bb)
            base = jnp.where(has, cum + prev, base)
            out += [bb, base]
        return tuple(out)

    res = lax.fori_loop(g0, g0 + NGRP, gbody, init)
    return res[1:]


def _sc_body(z_hbm, out_hbm, row_a, row_b, hist1, hist2,
             sem_in_a, sem_in_b, sem_out_a, sem_out_b):
    wid = lax.axis_index("s") * NC + lax.axis_index("c")
    lane = lax.iota(jnp.int32, L)
    lane16 = _shl(lane, 4)
    ones = jnp.ones((L,), jnp.int32)
    bufs = (row_a, row_b)
    sems_in = (sem_in_a, sem_in_b)
    sems_out = (sem_out_a, sem_out_b)
    base0 = wid * (ROWS_PER_W * COLS)

    def start_in(r):
        return pltpu.async_copy(
            z_hbm.at[pl.ds(base0 + r * COLS, COLS)], bufs[r % 2],
            sems_in[r % 2])

    def start_out(r):
        return pltpu.async_copy(
            bufs[r % 2], out_hbm.at[pl.ds(base0 + r * COLS, COLS)],
            sems_out[r % 2])

    def compute_row(row_v):
        _clear(hist1, NBKT * L)

        @plsc.parallel_loop(0, NVEC, unroll=UNROLL)
        def p1(i):
            uk = _ukey(row_v[pl.ds(i * L, L)])
            idx = lax.bitwise_or(_shl(_srl(uk, 24), 4), lane)
            plsc.addupdate_scatter(hist1, [idx], ones)

        b1p, base1p, b1n, base1n = _vsearch(
            hist1, lane16, (jnp.int32(R_POS), jnp.int32(R_NEG)), 0)
        r2p = jnp.int32(R_POS) - base1p
        r2n = jnp.int32(R_NEG) - base1n

        _clear(hist2, 2 * NBKT * L)

        @plsc.parallel_loop(0, NVEC, unroll=UNROLL)
        def p2(i):
            uk = _ukey(row_v[pl.ds(i * L, L)])
            top8 = _srl(uk, 24)
            mp = top8 == b1p
            mn = top8 == b1n
            b2 = lax.bitwise_and(_srl(uk, 16), FF)
            bucket = jnp.where(mp, b2, b2 + NBKT)
            idx = lax.bitwise_or(_shl(bucket, 4), lane)
            plsc.addupdate_scatter(hist2, [idx], ones,
                                   mask=jnp.logical_or(mp, mn))

        b2p, base2p = _vsearch(hist2, lane16, (r2p,), 0)
        b2n, base2n = _vsearch(hist2, lane16, (r2n,), NGRP)
        r3p = r2p - base2p
        r3n = r2n - base2n
        pfx16p = lax.bitwise_or(_shl(b1p, 8), b2p)
        pfx16n = lax.bitwise_or(_shl(b1n, 8), b2n)

        _clear(hist2, 2 * NBKT * L)

        @plsc.parallel_loop(0, NVEC, unroll=UNROLL)
        def p3(i):
            uk = _ukey(row_v[pl.ds(i * L, L)])
            top16 = _srl(uk, 16)
            mp = top16 == pfx16p
            mn = top16 == pfx16n
            b3 = lax.bitwise_and(_srl(uk, 8), FF)
            bucket = jnp.where(mp, b3, b3 + NBKT)
            idx = lax.bitwise_or(_shl(bucket, 4), lane)
            plsc.addupdate_scatter(hist2, [idx], ones,
                                   mask=jnp.logical_or(mp, mn))

        b3p, _ = _vsearch(hist2, lane16, (r3p,), 0)
        b3n, _ = _vsearch(hist2, lane16, (r3n,), NGRP)

        utp = lax.bitwise_or(_shl(pfx16p, 16), _shl(b3p, 8))
        utn = lax.bitwise_or(lax.bitwise_or(_shl(pfx16n, 16), _shl(b3n, 8)),
                             FF)
        stp = lax.bitwise_xor(utp, INT_MIN)
        stn = lax.bitwise_xor(utn, INT_MIN)

        @plsc.parallel_loop(0, NVEC, unroll=UNROLL)
        def p4(i):
            zv = row_v[pl.ds(i * L, L)]
            sk = lax.bitwise_xor(_ukey(zv), INT_MIN)
            keep = jnp.logical_or(sk >= stp, sk <= stn)
            row_v[pl.ds(i * L, L)] = jnp.where(keep, zv, jnp.float32(0.0))

    def do_row(r, _):
        off = base0 + r * COLS
        pltpu.sync_copy(z_hbm.at[pl.ds(off, COLS)], row_a)
        compute_row(row_a)
        pltpu.sync_copy(row_a, out_hbm.at[pl.ds(off, COLS)])
        return 0

    lax.fori_loop(0, ROWS_PER_W, do_row, 0)


@jax.jit
def _run(zf):
    mesh = plsc.VectorSubcoreMesh(core_axis_name="c", subcore_axis_name="s",
                                  num_cores=NC, num_subcores=NS)
    f = pl.kernel(
        _sc_body,
        out_type=jax.ShapeDtypeStruct((ROWS * COLS,), jnp.float32),
        mesh=mesh,
        compiler_params=pltpu.CompilerParams(needs_layout_passes=False),
        scratch_types=[
            pltpu.VMEM((COLS,), jnp.float32),
            pltpu.VMEM((COLS,), jnp.float32),
            pltpu.VMEM((NBKT * L,), jnp.int32),
            pltpu.VMEM((2 * NBKT * L,), jnp.int32),
            pltpu.SemaphoreType.DMA,
            pltpu.SemaphoreType.DMA,
            pltpu.SemaphoreType.DMA,
            pltpu.SemaphoreType.DMA,
        ],
    )
    return f(zf)


def kernel(z):
    return _run(z.reshape(-1)).reshape(ROWS, COLS)


# speedup vs baseline: 1.1542x; 1.1542x over previous
import jax
import jax.numpy as jnp
from jax import lax
from jax.experimental import pallas as pl
from jax.experimental.pallas import tpu as pltpu
from jax.experimental.pallas import tpu_sc as plsc

ROWS = 128
COLS = 32768
K = COLS // 4
R_POS = COLS - K + 1
R_NEG = K
L = 16
NVEC = COLS // L
NBKT = 256
NGRP = NBKT // L
NC, NS = 2, 16
NW = NC * NS
ROWS_PER_W = ROWS // NW
UNROLL = 8

FF = jnp.int32(0xFF)
FF0 = jnp.int32(0xFF0)


def _srl(x, n):
    return lax.shift_right_logical(x, jnp.int32(n))


def _shl(x, n):
    return lax.shift_left(x, jnp.int32(n))


def _clear(hist, nvals):
    zeros = jnp.zeros((L,), jnp.int32)

    @plsc.parallel_loop(0, nvals // L, unroll=UNROLL)
    def body(b):
        hist[pl.ds(b * L, L)] = zeros


def _vsearch(hist, targets, bucket_off, bucket_of_p):
    lane = lax.iota(jnp.int32, L)
    z = jnp.int32(0)
    init = (z,) + sum(((z, z) for _ in targets), ())

    def gbody(g, carry):
        cum = carry[0]
        p = _shl(g, 4) + lane
        flat = _shl(bucket_of_p(p) + bucket_off, 4)
        acc = jnp.zeros((L,), jnp.int32)
        for j in range(L):
            acc = acc + plsc.load_gather(hist, [flat + j])
        cs = plsc.cumsum(acc)
        tot = jnp.max(cs)
        cum2 = cum + tot
        excl = cs - acc
        out = [cum2]
        for t, r in enumerate(targets):
            bb, base = carry[1 + 2 * t], carry[2 + 2 * t]
            hit_vec = (cum + cs) >= r
            has = jnp.logical_and(cum < r, cum2 >= r)
            idx_splat = plsc.all_reduce_ffs(hit_vec)
            prev = jnp.max(jnp.where(lane == idx_splat, excl, z))
            bb = jnp.where(has, _shl(g, 4) + jnp.max(idx_splat), bb)
            base = jnp.where(has, cum + prev, base)
            out += [bb, base]
        return tuple(out)

    res = lax.fori_loop(0, NGRP, gbody, init)
    return res[1:]


def _map_l1(p):
    return jnp.where(p < 128, 255 - p, p - 128)


def _sc_body(z_hbm, out_hbm, row_v, hist1, hist2):
    wid = lax.axis_index("s") * NC + lax.axis_index("c")
    lane = lax.iota(jnp.int32, L)
    ones = jnp.ones((L,), jnp.int32)
    base0 = wid * (ROWS_PER_W * COLS)

    def compute_row():
        _clear(hist1, NBKT * L)

        @plsc.parallel_loop(0, NVEC, unroll=UNROLL)
        def p1(i):
            bits = lax.bitcast_convert_type(row_v[pl.ds(i * L, L)], jnp.int32)
            idx = lax.bitwise_or(lax.bitwise_and(_srl(bits, 20), FF0), lane)
            plsc.addupdate_scatter(hist1, [idx], ones)

        pos1p, base1p, pos1n, base1n = _vsearch(
            hist1, (jnp.int32(R_POS), jnp.int32(R_NEG)), 0, _map_l1)
        b1p = _map_l1(pos1p)
        b1n = _map_l1(pos1n)
        negp = pos1p < 128
        negn = pos1n < 128
        r2p = jnp.int32(R_POS) - base1p
        r2n = jnp.int32(R_NEG) - base1n

        def map_desc(desc):
            return lambda p: jnp.where(desc, 255 - p, p)

        _clear(hist2, 2 * NBKT * L)

        @plsc.parallel_loop(0, NVEC, unroll=UNROLL)
        def p2(i):
            bits = lax.bitcast_convert_type(row_v[pl.ds(i * L, L)], jnp.int32)
            top8 = _srl(bits, 24)
            mp = top8 == b1p
            mn = top8 == b1n
            b2 = lax.bitwise_and(_srl(bits, 16), FF)
            bucket = jnp.where(mp, b2, b2 + NBKT)
            idx = lax.bitwise_or(_shl(bucket, 4), lane)
            plsc.addupdate_scatter(hist2, [idx], ones,
                                   mask=jnp.logical_or(mp, mn))

        pos2p, base2p = _vsearch(hist2, (r2p,), 0, map_desc(negp))
        pos2n, base2n = _vsearch(hist2, (r2n,), NBKT, map_desc(negn))
        b2p = jnp.where(negp, 255 - pos2p, pos2p)
        b2n = jnp.where(negn, 255 - pos2n, pos2n)
        r3p = r2p - base2p
        r3n = r2n - base2n
        pfx16p = lax.bitwise_or(_shl(b1p, 8), b2p)
        pfx16n = lax.bitwise_or(_shl(b1n, 8), b2n)

        _clear(hist2, 2 * NBKT * L)

        @plsc.parallel_loop(0, NVEC, unroll=UNROLL)
        def p3(i):
            bits = lax.bitcast_convert_type(row_v[pl.ds(i * L, L)], jnp.int32)
            top16 = _srl(bits, 16)
            mp = top16 == pfx16p
            mn = top16 == pfx16n
            b3 = lax.bitwise_and(_srl(bits, 8), FF)
            bucket = jnp.where(mp, b3, b3 + NBKT)
            idx = lax.bitwise_or(_shl(bucket, 4), lane)
            plsc.addupdate_scatter(hist2, [idx], ones,
                                   mask=jnp.logical_or(mp, mn))

        pos3p, _ = _vsearch(hist2, (r3p,), 0, map_desc(negp))
        pos3n, _ = _vsearch(hist2, (r3n,), NBKT, map_desc(negn))
        b3p = jnp.where(negp, 255 - pos3p, pos3p)
        b3n = jnp.where(negn, 255 - pos3n, pos3n)

        tp_bits = lax.bitwise_or(
            lax.bitwise_or(_shl(pfx16p, 16), _shl(b3p, 8)),
            jnp.where(negp, FF, jnp.int32(0)))
        tn_bits = lax.bitwise_or(
            lax.bitwise_or(_shl(pfx16n, 16), _shl(b3n, 8)),
            jnp.where(negn, jnp.int32(0), FF))
        tp_vec = lax.bitcast_convert_type(
            jnp.broadcast_to(tp_bits, (L,)), jnp.float32)
        tn_vec = lax.bitcast_convert_type(
            jnp.broadcast_to(tn_bits, (L,)), jnp.float32)

        @plsc.parallel_loop(0, NVEC, unroll=UNROLL)
        def p4(i):
            zv = row_v[pl.ds(i * L, L)]
            keep = jnp.logical_or(zv >= tp_vec, zv <= tn_vec)
            row_v[pl.ds(i * L, L)] = jnp.where(keep, zv, jnp.float32(0.0))

    def do_row(r, _):
        off = base0 + r * COLS
        pltpu.sync_copy(z_hbm.at[pl.ds(off, COLS)], row_v)
        compute_row()
        pltpu.sync_copy(row_v, out_hbm.at[pl.ds(off, COLS)])
        return 0

    lax.fori_loop(0, ROWS_PER_W, do_row, 0)


@jax.jit
def _run(zf):
    mesh = plsc.VectorSubcoreMesh(core_axis_name="c", subcore_axis_name="s",
                                  num_cores=NC, num_subcores=NS)
    f = pl.kernel(
        _sc_body,
        out_type=jax.ShapeDtypeStruct((ROWS * COLS,), jnp.float32),
        mesh=mesh,
        compiler_params=pltpu.CompilerParams(needs_layout_passes=False),
        scratch_types=[
            pltpu.VMEM((COLS,), jnp.float32),
            pltpu.VMEM((NBKT * L,), jnp.int32),
            pltpu.VMEM((2 * NBKT * L,), jnp.int32),
        ],
    )
    return f(zf)


def kernel(z):
    return _run(z.reshape(-1)).reshape(ROWS, COLS)


# speedup vs baseline: 1.2804x; 1.1094x over previous
import jax
import jax.numpy as jnp
from jax import lax
from jax.experimental import pallas as pl
from jax.experimental.pallas import tpu as pltpu
from jax.experimental.pallas import tpu_sc as plsc

ROWS = 128
COLS = 32768
K = COLS // 4
R_POS = COLS - K + 1
R_NEG = K
L = 16
NVEC = COLS // L
NBKT = 256
NGRP = NBKT // L
NC, NS = 2, 16
NW = NC * NS
ROWS_PER_W = ROWS // NW
UNROLL = 8

FF = jnp.int32(0xFF)
FF0 = jnp.int32(0xFF0)


def _srl(x, n):
    return lax.shift_right_logical(x, jnp.int32(n))


def _shl(x, n):
    return lax.shift_left(x, jnp.int32(n))


def _clear(hist, nvals):
    zeros = jnp.zeros((L,), jnp.int32)

    @plsc.parallel_loop(0, nvals // L, unroll=UNROLL)
    def body(b):
        hist[pl.ds(b * L, L)] = zeros


def _vsearch(hist, targets, bucket_off, bucket_of_p, lane_major=True):
    lane = lax.iota(jnp.int32, L)
    z = jnp.int32(0)
    init = (z,) + sum(((z, z) for _ in targets), ())

    def gbody(g, carry):
        cum = carry[0]
        p = _shl(g, 4) + lane
        if lane_major:
            flat = _shl(bucket_of_p(p) + bucket_off, 4)
            acc = jnp.zeros((L,), jnp.int32)
            for j in range(L):
                acc = acc + plsc.load_gather(hist, [flat + j])
        else:
            acc = plsc.load_gather(hist, [bucket_of_p(p) + bucket_off])
        cs = plsc.cumsum(acc)
        tot = jnp.max(cs)
        cum2 = cum + tot
        excl = cs - acc
        out = [cum2]
        for t, r in enumerate(targets):
            bb, base = carry[1 + 2 * t], carry[2 + 2 * t]
            hit_vec = (cum + cs) >= r
            has = jnp.logical_and(cum < r, cum2 >= r)
            idx_splat = plsc.all_reduce_ffs(hit_vec)
            prev = jnp.max(jnp.where(lane == idx_splat, excl, z))
            bb = jnp.where(has, _shl(g, 4) + jnp.max(idx_splat), bb)
            base = jnp.where(has, cum + prev, base)
            out += [bb, base]
        return tuple(out)

    res = lax.fori_loop(0, NGRP, gbody, init)
    return res[1:]


def _map_l1(p):
    return jnp.where(p < 128, 255 - p, p - 128)


def _sc_body(z_hbm, out_hbm, row_v, hist1, hist2):
    wid = lax.axis_index("s") * NC + lax.axis_index("c")
    lane = lax.iota(jnp.int32, L)
    ones = jnp.ones((L,), jnp.int32)
    base0 = wid * (ROWS_PER_W * COLS)

    def compute_row():
        _clear(hist1, NBKT * L)

        @plsc.parallel_loop(0, NVEC, unroll=UNROLL)
        def p1(i):
            bits = lax.bitcast_convert_type(row_v[pl.ds(i * L, L)], jnp.int32)
            idx = lax.bitwise_or(lax.bitwise_and(_srl(bits, 20), FF0), lane)
            plsc.addupdate_scatter(hist1, [idx], ones)

        pos1p, base1p, pos1n, base1n = _vsearch(
            hist1, (jnp.int32(R_POS), jnp.int32(R_NEG)), 0, _map_l1)
        b1p = _map_l1(pos1p)
        b1n = _map_l1(pos1n)
        negp = pos1p < 128
        negn = pos1n < 128
        r2p = jnp.int32(R_POS) - base1p
        r2n = jnp.int32(R_NEG) - base1n

        def map_desc(desc):
            return lambda p: jnp.where(desc, 255 - p, p)

        _clear(hist2, 2 * NBKT)

        @plsc.parallel_loop(0, NVEC, unroll=UNROLL)
        def p2(i):
            bits = lax.bitcast_convert_type(row_v[pl.ds(i * L, L)], jnp.int32)
            top8 = _srl(bits, 24)
            mp = top8 == b1p
            mn = top8 == b1n
            b2 = lax.bitwise_and(_srl(bits, 16), FF)
            bucket = jnp.where(mp, b2, b2 + NBKT)
            plsc.addupdate_scatter(hist2, [bucket], ones,
                                   mask=jnp.logical_or(mp, mn))

        pos2p, base2p = _vsearch(hist2, (r2p,), 0, map_desc(negp), lane_major=False)
        pos2n, base2n = _vsearch(hist2, (r2n,), NBKT, map_desc(negn), lane_major=False)
        b2p = jnp.where(negp, 255 - pos2p, pos2p)
        b2n = jnp.where(negn, 255 - pos2n, pos2n)
        r3p = r2p - base2p
        r3n = r2n - base2n
        pfx16p = lax.bitwise_or(_shl(b1p, 8), b2p)
        pfx16n = lax.bitwise_or(_shl(b1n, 8), b2n)

        _clear(hist2, 2 * NBKT)

        @plsc.parallel_loop(0, NVEC, unroll=UNROLL)
        def p3(i):
            bits = lax.bitcast_convert_type(row_v[pl.ds(i * L, L)], jnp.int32)
            top16 = _srl(bits, 16)
            mp = top16 == pfx16p
            mn = top16 == pfx16n
            b3 = lax.bitwise_and(_srl(bits, 8), FF)
            bucket = jnp.where(mp, b3, b3 + NBKT)
            plsc.addupdate_scatter(hist2, [bucket], ones,
                                   mask=jnp.logical_or(mp, mn))

        pos3p, _ = _vsearch(hist2, (r3p,), 0, map_desc(negp), lane_major=False)
        pos3n, _ = _vsearch(hist2, (r3n,), NBKT, map_desc(negn), lane_major=False)
        b3p = jnp.where(negp, 255 - pos3p, pos3p)
        b3n = jnp.where(negn, 255 - pos3n, pos3n)

        tp_bits = lax.bitwise_or(
            lax.bitwise_or(_shl(pfx16p, 16), _shl(b3p, 8)),
            jnp.where(negp, FF, jnp.int32(0)))
        tn_bits = lax.bitwise_or(
            lax.bitwise_or(_shl(pfx16n, 16), _shl(b3n, 8)),
            jnp.where(negn, jnp.int32(0), FF))
        tp_vec = lax.bitcast_convert_type(
            jnp.broadcast_to(tp_bits, (L,)), jnp.float32)
        tn_vec = lax.bitcast_convert_type(
            jnp.broadcast_to(tn_bits, (L,)), jnp.float32)

        @plsc.parallel_loop(0, NVEC, unroll=UNROLL)
        def p4(i):
            zv = row_v[pl.ds(i * L, L)]
            keep = jnp.logical_or(zv >= tp_vec, zv <= tn_vec)
            row_v[pl.ds(i * L, L)] = jnp.where(keep, zv, jnp.float32(0.0))

    def do_row(r, _):
        off = base0 + r * COLS
        pltpu.sync_copy(z_hbm.at[pl.ds(off, COLS)], row_v)
        compute_row()
        pltpu.sync_copy(row_v, out_hbm.at[pl.ds(off, COLS)])
        return 0

    lax.fori_loop(0, ROWS_PER_W, do_row, 0)


@jax.jit
def _run(zf):
    mesh = plsc.VectorSubcoreMesh(core_axis_name="c", subcore_axis_name="s",
                                  num_cores=NC, num_subcores=NS)
    f = pl.kernel(
        _sc_body,
        out_type=jax.ShapeDtypeStruct((ROWS * COLS,), jnp.float32),
        mesh=mesh,
        compiler_params=pltpu.CompilerParams(needs_layout_passes=False),
        scratch_types=[
            pltpu.VMEM((COLS,), jnp.float32),
            pltpu.VMEM((NBKT * L,), jnp.int32),
            pltpu.VMEM((2 * NBKT,), jnp.int32),
        ],
    )
    return f(zf)


def kernel(z):
    return _run(z.reshape(-1)).reshape(ROWS, COLS)
